# Initial kernel scaffold; baseline (speedup 1.0000x reference)
import jax
import jax.numpy as jnp
from jax.experimental import pallas as pl


def kernel(edge_attr, edge_embedding_weight):
    raise NotImplementedError("write your pallas kernel here")


# speedup vs baseline: 5.7803x; 5.7803x over previous
import functools

import jax
import jax.numpy as jnp
from jax import lax
from jax.experimental import pallas as pl
from jax.experimental.pallas import tpu as pltpu
from jax.experimental.pallas import tpu_sc as plsc

_NC = 2
_NS = 16
_L = 16


def _build(n_edges, vocab, emb, chunk, interpret=False):
    nw = _NC * _NS
    b_per_w = n_edges // nw
    n_chunks = b_per_w // chunk
    groups = chunk // _L

    mesh = plsc.VectorSubcoreMesh(core_axis_name="c", subcore_axis_name="s")

    @functools.partial(
        pl.kernel,
        out_type=jax.ShapeDtypeStruct((n_edges * emb,), jnp.float32),
        mesh=mesh,
        scratch_types=[
            pltpu.VMEM((vocab * emb,), jnp.float32),
            pltpu.VMEM((chunk,), jnp.int32),
            pltpu.VMEM((chunk * emb,), jnp.float32),
        ],
        interpret=interpret,
        compiler_params=pltpu.CompilerParams(needs_layout_passes=False),
    )
    def k(idx_hbm, table_hbm, out_hbm, table_v, idx_v, rows_v):
        wid = lax.axis_index("s") * _NC + lax.axis_index("c")
        pltpu.sync_copy(table_hbm, table_v)
        base_w = pl.multiple_of(wid * b_per_w, chunk)
        lane = lax.iota(jnp.int32, _L)

        def chunk_body(c, carry):
            base = pl.multiple_of(base_w + c * chunk, chunk)
            pltpu.sync_copy(idx_hbm.at[pl.ds(base, chunk)], idx_v)

            def group_body(g, carry2):
                off = pl.multiple_of(g * _L, _L)
                idxv = idx_v[pl.ds(off, _L)]
                srcb = idxv * emb
                dstb = off * emb + lane * emb
                for d in range(emb):
                    col = plsc.load_gather(table_v, [srcb + d])
                    plsc.store_scatter(rows_v, [dstb + d], col)
                return carry2

            lax.fori_loop(0, groups, group_body, 0)
            pltpu.sync_copy(
                rows_v, out_hbm.at[pl.ds(base * emb, chunk * emb)]
            )
            return carry

        lax.fori_loop(0, n_chunks, chunk_body, 0)

    return k


def kernel(edge_attr, edge_embedding_weight):
    n_edges = edge_attr.shape[0]
    vocab, emb = edge_embedding_weight.shape
    k = _build(n_edges, vocab, emb, chunk=2000)
    flat = k(edge_attr.astype(jnp.int32), edge_embedding_weight.reshape(-1))
    return flat.reshape(n_edges, emb)
